# baseline (speedup 1.0000x reference)
import jax
import jax.numpy as jnp
from jax.experimental import pallas as pl
from jax.experimental.pallas import tpu as pltpu

_K = 16
_ALPHA = 3.0


def _nv_kernel(x1_ref, x2_ref, w1t_ref, b1_ref, w2t_ref, b2_ref,
               nv1_ref, nv2_ref):
    nv1_ref[...] = jnp.tanh(
        _ALPHA * (jnp.dot(x1_ref[...], w1t_ref[...],
                          preferred_element_type=jnp.float32) + b1_ref[...]))
    nv2_ref[...] = jnp.tanh(
        _ALPHA * (jnp.dot(x2_ref[...], w2t_ref[...],
                          preferred_element_type=jnp.float32) + b2_ref[...]))


def _noise_at(p):
    u32 = jnp.uint32
    ks0 = u32(0)
    ks1 = u32(1)
    ks2 = u32(0x1BD11BDB)
    ks = (ks0, ks1, ks2)
    x0 = jnp.zeros_like(p, dtype=u32) + ks0
    x1 = p.astype(u32) + ks1

    def rotl(v, d):
        return jnp.left_shift(v, u32(d)) | jnp.right_shift(v, u32(32 - d))

    rot_groups = ((13, 15, 26, 6), (17, 29, 16, 24))
    for g in range(5):
        for r in rot_groups[g % 2]:
            x0 = x0 + x1
            x1 = rotl(x1, r)
            x1 = x0 ^ x1
        x0 = x0 + ks[(g + 1) % 3]
        x1 = x1 + ks[(g + 2) % 3] + u32(g + 1)
    bits = x0 ^ x1
    f = jax.lax.bitcast_convert_type(
        jnp.right_shift(bits, u32(9)) | u32(0x3F800000), jnp.float32)
    return (f - 1.0) * 0.01


def _main_kernel(n, rb, nv1b_ref, nv2b_ref, nv1_ref, nv2_ref,
                 adj_ref, v_ref, ew_ref):
    i = pl.program_id(0)
    dn = (((1,), (1,)), ((), ()))
    col = jax.lax.broadcasted_iota(jnp.int32, (rb, n), 1)
    row = i * rb + jax.lax.broadcasted_iota(jnp.int32, (rb, 1), 0)

    a = (jax.lax.dot_general(nv1b_ref[...], nv2_ref[...], dn,
                             preferred_element_type=jnp.float32)
         - jax.lax.dot_general(nv2b_ref[...], nv1_ref[...], dn,
                               preferred_element_type=jnp.float32))
    adj = jax.nn.relu(jnp.tanh(_ALPHA * a))
    t = adj + _noise_at(row * n + col)

    neg_inf = jnp.float32(-jnp.inf)

    W = 128
    C = (n + W - 1) // W
    pad = C * W - n
    if pad:
        tp = jnp.concatenate(
            [t, jnp.full((rb, pad), neg_inf, jnp.float32)], axis=1)
    else:
        tp = t
    t3 = tp.reshape(rb, C, W)
    liota = jax.lax.broadcasted_iota(jnp.int32, (rb, C, W), 2)
    ciota = jax.lax.broadcasted_iota(jnp.int32, (rb, C), 1)

    def extract4(tt, mask):
        vs, ls = [], []
        for _ in range(4):
            mv = jnp.max(tt, axis=2)
            lv = jnp.min(jnp.where(tt == mv[:, :, None], liota, W),
                         axis=2).astype(jnp.int32)
            lkill = lv if mask is None else jnp.where(mask, lv, W)
            tt = jnp.where(liota == lkill[:, :, None], neg_inf, tt)
            vs.append(mv)
            ls.append(lv)
        return tt, vs, ls

    t3, vs, ls = extract4(t3, None)
    state = (t3, vs[0], vs[1], vs[2], vs[3], ls[0], ls[1], ls[2], ls[3])

    def refill(st):
        tt, v1, v2, v3, v4, l1, l2, l3, l4 = st
        ex = v1 == neg_inf
        tt, nv, nl = extract4(tt, ex)
        ov = (v1, v2, v3, v4)
        ol = (l1, l2, l3, l4)
        v = [jnp.where(ex, nv[j], ov[j]) for j in range(4)]
        l = [jnp.where(ex, nl[j], ol[j]) for j in range(4)]
        return (tt, v[0], v[1], v[2], v[3], l[0], l[1], l[2], l[3])

    idx_cols = []
    m_cols = []
    for k in range(_K):
        t3, v1, v2, v3, v4, l1, l2, l3, l4 = state
        m = jnp.max(v1, axis=1, keepdims=True)
        csel = jnp.min(jnp.where(v1 == m, ciota, C), axis=1,
                       keepdims=True).astype(jnp.int32)
        hit = ciota == csel
        lsel = jnp.sum(jnp.where(hit, l1, 0), axis=1, keepdims=True)
        idx_cols.append(csel * W + lsel)
        m_cols.append(m)
        v1 = jnp.where(hit, v2, v1)
        v2 = jnp.where(hit, v3, v2)
        v3 = jnp.where(hit, v4, v3)
        v4 = jnp.where(hit, neg_inf, v4)
        l1 = jnp.where(hit, l2, l1)
        l2 = jnp.where(hit, l3, l2)
        l3 = jnp.where(hit, l4, l3)
        state = (t3, v1, v2, v3, v4, l1, l2, l3, l4)
        if k < _K - 1:
            state = jax.lax.cond(jnp.any(v1 == neg_inf), refill,
                                 lambda st: st, state)

    idx = jnp.concatenate(idx_cols, axis=1)
    m = jnp.concatenate(m_cols, axis=1)
    m16 = m_cols[-1]
    idx16 = idx_cols[-1]
    keep = jnp.logical_or(t > m16,
                          jnp.logical_and(t == m16, col <= idx16))

    eye = (col == row).astype(jnp.float32)
    adj_ref[...] = jnp.where(keep, adj, 0.0) + eye
    w = m - _noise_at(row * n + idx) + (idx == row).astype(jnp.float32)
    v_ref[...] = jnp.concatenate([idx, row], axis=1)
    ew_ref[...] = jnp.concatenate(
        [w, jnp.ones((rb, 1), dtype=jnp.float32)], axis=1)


def _row_block(n):
    for rb in (80, 64, 48, 40, 32, 24, 16, 8):
        if n % rb == 0:
            return rb
    return 1


def kernel(idx, node_emb1, node_emb2, emb1_w, emb2_w, lin1_w, lin1_b,
           lin2_w, lin2_b):
    n = idx.shape[0]
    dim = emb1_w.shape[1]
    x1 = jnp.take(emb1_w, idx, axis=0)
    x2 = jnp.take(emb2_w, idx, axis=0)

    nv1, nv2 = pl.pallas_call(
        _nv_kernel,
        out_shape=(jax.ShapeDtypeStruct((n, dim), jnp.float32),
                   jax.ShapeDtypeStruct((n, dim), jnp.float32)),
    )(x1, x2, lin1_w.T, lin1_b[None, :], lin2_w.T, lin2_b[None, :])

    rb = _row_block(n)
    grid = n // rb
    adj, v, ew = pl.pallas_call(
        lambda *refs: _main_kernel(n, rb, *refs),
        grid=(grid,),
        in_specs=[
            pl.BlockSpec((rb, dim), lambda i: (i, 0)),
            pl.BlockSpec((rb, dim), lambda i: (i, 0)),
            pl.BlockSpec((n, dim), lambda i: (0, 0)),
            pl.BlockSpec((n, dim), lambda i: (0, 0)),
        ],
        out_specs=[
            pl.BlockSpec((rb, n), lambda i: (i, 0)),
            pl.BlockSpec((rb, _K + 1), lambda i: (i, 0)),
            pl.BlockSpec((rb, _K + 1), lambda i: (i, 0)),
        ],
        out_shape=(jax.ShapeDtypeStruct((n, n), jnp.float32),
                   jax.ShapeDtypeStruct((n, _K + 1), jnp.int32),
                   jax.ShapeDtypeStruct((n, _K + 1), jnp.float32)),
    )(nv1, nv2, nv1, nv2)

    u = jnp.repeat(jnp.arange(n, dtype=jnp.int32), _K + 1)
    return adj, ew.reshape(-1), u, v.reshape(-1)


# speedup vs baseline: 5.4747x; 5.4747x over previous
import jax
import jax.numpy as jnp
from jax.experimental import pallas as pl
from jax.experimental.pallas import tpu as pltpu

_K = 16
_ALPHA = 3.0


def _nv_kernel(x1_ref, x2_ref, w1t_ref, b1_ref, w2t_ref, b2_ref,
               nv1_ref, nv2_ref):
    nv1_ref[...] = jnp.tanh(
        _ALPHA * (jnp.dot(x1_ref[...], w1t_ref[...],
                          preferred_element_type=jnp.float32) + b1_ref[...]))
    nv2_ref[...] = jnp.tanh(
        _ALPHA * (jnp.dot(x2_ref[...], w2t_ref[...],
                          preferred_element_type=jnp.float32) + b2_ref[...]))


def _noise_at(p):
    u32 = jnp.uint32
    ks0 = u32(0)
    ks1 = u32(1)
    ks2 = u32(0x1BD11BDB)
    ks = (ks0, ks1, ks2)
    x0 = jnp.zeros_like(p, dtype=u32) + ks0
    x1 = p.astype(u32) + ks1

    def rotl(v, d):
        return jnp.left_shift(v, u32(d)) | jnp.right_shift(v, u32(32 - d))

    rot_groups = ((13, 15, 26, 6), (17, 29, 16, 24))
    for g in range(5):
        for r in rot_groups[g % 2]:
            x0 = x0 + x1
            x1 = rotl(x1, r)
            x1 = x0 ^ x1
        x0 = x0 + ks[(g + 1) % 3]
        x1 = x1 + ks[(g + 2) % 3] + u32(g + 1)
    bits = x0 ^ x1
    f = jax.lax.bitcast_convert_type(
        jnp.right_shift(bits, u32(9)) | u32(0x3F800000), jnp.float32)
    return (f - 1.0) * 0.01


def _main_kernel(n, rb, nv1b_ref, nv2b_ref, nv1_ref, nv2_ref,
                 adj_ref, v_ref, ew_ref):
    i = pl.program_id(0)
    dn = (((1,), (1,)), ((), ()))
    col = jax.lax.broadcasted_iota(jnp.int32, (rb, n), 1)
    row = i * rb + jax.lax.broadcasted_iota(jnp.int32, (rb, 1), 0)

    a = (jax.lax.dot_general(nv1b_ref[...], nv2_ref[...], dn,
                             preferred_element_type=jnp.float32)
         - jax.lax.dot_general(nv2b_ref[...], nv1_ref[...], dn,
                               preferred_element_type=jnp.float32))
    adj = jax.nn.relu(jnp.tanh(_ALPHA * a))
    t = adj + _noise_at(row * n + col)

    neg_inf = jnp.float32(-jnp.inf)

    nseg = 4
    segw = -(-n // (nseg * 128)) * 128
    bounds = [min(s * segw, n) for s in range(nseg + 1)]
    cand_v = []
    cand_f = []
    for s in range(nseg):
        a, b = bounds[s], bounds[s + 1]
        if a == b:
            continue
        ts = t[:, a:b]
        cs = col[:, a:b]
        for _ in range(_K):
            m = jnp.max(ts, axis=1, keepdims=True)
            fk = jnp.min(jnp.where(ts == m, cs, n), axis=1,
                         keepdims=True).astype(jnp.int32)
            ts = jnp.where(cs == fk, neg_inf, ts)
            cand_v.append(m)
            cand_f.append(fk)
    pv = jnp.concatenate(cand_v, axis=1)
    pf = jnp.concatenate(cand_f, axis=1)
    idx_cols = []
    m_cols = []
    for _ in range(_K):
        m = jnp.max(pv, axis=1, keepdims=True)
        fk = jnp.min(jnp.where(pv == m, pf, n), axis=1,
                     keepdims=True).astype(jnp.int32)
        pv = jnp.where(pf == fk, neg_inf, pv)
        idx_cols.append(fk)
        m_cols.append(m)
    idx = jnp.concatenate(idx_cols, axis=1)
    m = jnp.concatenate(m_cols, axis=1)
    m16 = m_cols[-1]
    idx16 = idx_cols[-1]
    keep = jnp.logical_or(t > m16,
                          jnp.logical_and(t == m16, col <= idx16))

    eye = (col == row).astype(jnp.float32)
    adj_ref[...] = jnp.where(keep, adj, 0.0) + eye
    w = m - _noise_at(row * n + idx) + (idx == row).astype(jnp.float32)
    v_ref[...] = jnp.concatenate([idx, row], axis=1)
    ew_ref[...] = jnp.concatenate(
        [w, jnp.ones((rb, 1), dtype=jnp.float32)], axis=1)


def _row_block(n):
    for rb in (80, 64, 48, 40, 32, 24, 16, 8):
        if n % rb == 0:
            return rb
    return 1


def kernel(idx, node_emb1, node_emb2, emb1_w, emb2_w, lin1_w, lin1_b,
           lin2_w, lin2_b):
    n = idx.shape[0]
    dim = emb1_w.shape[1]
    x1 = jnp.take(emb1_w, idx, axis=0)
    x2 = jnp.take(emb2_w, idx, axis=0)

    nv1, nv2 = pl.pallas_call(
        _nv_kernel,
        out_shape=(jax.ShapeDtypeStruct((n, dim), jnp.float32),
                   jax.ShapeDtypeStruct((n, dim), jnp.float32)),
    )(x1, x2, lin1_w.T, lin1_b[None, :], lin2_w.T, lin2_b[None, :])

    rb = _row_block(n)
    grid = n // rb
    adj, v, ew = pl.pallas_call(
        lambda *refs: _main_kernel(n, rb, *refs),
        grid=(grid,),
        in_specs=[
            pl.BlockSpec((rb, dim), lambda i: (i, 0)),
            pl.BlockSpec((rb, dim), lambda i: (i, 0)),
            pl.BlockSpec((n, dim), lambda i: (0, 0)),
            pl.BlockSpec((n, dim), lambda i: (0, 0)),
        ],
        out_specs=[
            pl.BlockSpec((rb, n), lambda i: (i, 0)),
            pl.BlockSpec((rb, _K + 1), lambda i: (i, 0)),
            pl.BlockSpec((rb, _K + 1), lambda i: (i, 0)),
        ],
        out_shape=(jax.ShapeDtypeStruct((n, n), jnp.float32),
                   jax.ShapeDtypeStruct((n, _K + 1), jnp.int32),
                   jax.ShapeDtypeStruct((n, _K + 1), jnp.float32)),
    )(nv1, nv2, nv1, nv2)

    u = jnp.repeat(jnp.arange(n, dtype=jnp.int32), _K + 1)
    return adj, ew.reshape(-1), u, v.reshape(-1)


# speedup vs baseline: 5.9637x; 1.0893x over previous
import jax
import jax.numpy as jnp
from jax.experimental import pallas as pl
from jax.experimental.pallas import tpu as pltpu

_K = 16
_ALPHA = 3.0


def _nv_kernel(x1_ref, x2_ref, w1t_ref, b1_ref, w2t_ref, b2_ref,
               nv1_ref, nv2_ref):
    nv1_ref[...] = jnp.tanh(
        _ALPHA * (jnp.dot(x1_ref[...], w1t_ref[...],
                          preferred_element_type=jnp.float32) + b1_ref[...]))
    nv2_ref[...] = jnp.tanh(
        _ALPHA * (jnp.dot(x2_ref[...], w2t_ref[...],
                          preferred_element_type=jnp.float32) + b2_ref[...]))


def _noise_at(p):
    u32 = jnp.uint32
    ks0 = u32(0)
    ks1 = u32(1)
    ks2 = u32(0x1BD11BDB)
    ks = (ks0, ks1, ks2)
    x0 = jnp.zeros_like(p, dtype=u32) + ks0
    x1 = p.astype(u32) + ks1

    def rotl(v, d):
        return jnp.left_shift(v, u32(d)) | jnp.right_shift(v, u32(32 - d))

    rot_groups = ((13, 15, 26, 6), (17, 29, 16, 24))
    for g in range(5):
        for r in rot_groups[g % 2]:
            x0 = x0 + x1
            x1 = rotl(x1, r)
            x1 = x0 ^ x1
        x0 = x0 + ks[(g + 1) % 3]
        x1 = x1 + ks[(g + 2) % 3] + u32(g + 1)
    bits = x0 ^ x1
    f = jax.lax.bitcast_convert_type(
        jnp.right_shift(bits, u32(9)) | u32(0x3F800000), jnp.float32)
    return (f - 1.0) * 0.01


def _main_kernel(n, rb, nv1b_ref, nv2b_ref, nv1_ref, nv2_ref,
                 adj_ref, v_ref, ew_ref):
    i = pl.program_id(0)
    dn = (((1,), (1,)), ((), ()))
    col = jax.lax.broadcasted_iota(jnp.int32, (rb, n), 1)
    row = i * rb + jax.lax.broadcasted_iota(jnp.int32, (rb, 1), 0)

    a = (jax.lax.dot_general(nv1b_ref[...], nv2_ref[...], dn,
                             preferred_element_type=jnp.float32)
         - jax.lax.dot_general(nv2b_ref[...], nv1_ref[...], dn,
                               preferred_element_type=jnp.float32))
    adj = jax.nn.relu(jnp.tanh(_ALPHA * a))
    t = adj + _noise_at(row * n + col)

    neg_inf = jnp.float32(-jnp.inf)
    idx_cols = []
    m_cols = []
    for _ in range(_K):
        m = jnp.max(t, axis=1, keepdims=True)
        idxk = jnp.min(jnp.where(t == m, col, n), axis=1,
                       keepdims=True).astype(jnp.int32)
        t = jnp.where(col == idxk, neg_inf, t)
        idx_cols.append(idxk)
        m_cols.append(m)
    idx = jnp.concatenate(idx_cols, axis=1)
    m = jnp.concatenate(m_cols, axis=1)
    keep = t == neg_inf

    eye = (col == row).astype(jnp.float32)
    adj_ref[...] = jnp.where(keep, adj, 0.0) + eye
    w = m - _noise_at(row * n + idx) + (idx == row).astype(jnp.float32)
    v_ref[...] = jnp.concatenate([idx, row], axis=1)
    ew_ref[...] = jnp.concatenate(
        [w, jnp.ones((rb, 1), dtype=jnp.float32)], axis=1)


def _row_block(n):
    for rb in (80, 64, 48, 40, 32, 24, 16, 8):
        if n % rb == 0:
            return rb
    return 1


def kernel(idx, node_emb1, node_emb2, emb1_w, emb2_w, lin1_w, lin1_b,
           lin2_w, lin2_b):
    n = idx.shape[0]
    dim = emb1_w.shape[1]
    x1 = jnp.take(emb1_w, idx, axis=0)
    x2 = jnp.take(emb2_w, idx, axis=0)

    nv1, nv2 = pl.pallas_call(
        _nv_kernel,
        out_shape=(jax.ShapeDtypeStruct((n, dim), jnp.float32),
                   jax.ShapeDtypeStruct((n, dim), jnp.float32)),
    )(x1, x2, lin1_w.T, lin1_b[None, :], lin2_w.T, lin2_b[None, :])

    rb = _row_block(n)
    grid = n // rb
    adj, v, ew = pl.pallas_call(
        lambda *refs: _main_kernel(n, rb, *refs),
        grid=(grid,),
        in_specs=[
            pl.BlockSpec((rb, dim), lambda i: (i, 0)),
            pl.BlockSpec((rb, dim), lambda i: (i, 0)),
            pl.BlockSpec((n, dim), lambda i: (0, 0)),
            pl.BlockSpec((n, dim), lambda i: (0, 0)),
        ],
        out_specs=[
            pl.BlockSpec((rb, n), lambda i: (i, 0)),
            pl.BlockSpec((rb, _K + 1), lambda i: (i, 0)),
            pl.BlockSpec((rb, _K + 1), lambda i: (i, 0)),
        ],
        out_shape=(jax.ShapeDtypeStruct((n, n), jnp.float32),
                   jax.ShapeDtypeStruct((n, _K + 1), jnp.int32),
                   jax.ShapeDtypeStruct((n, _K + 1), jnp.float32)),
    )(nv1, nv2, nv1, nv2)

    u = jnp.repeat(jnp.arange(n, dtype=jnp.int32), _K + 1)
    return adj, ew.reshape(-1), u, v.reshape(-1)
